# baseline (speedup 1.0000x reference)
import math

import numpy as np
import jax
import jax.numpy as jnp
from jax import lax
from jax.experimental import pallas as pl
from jax.experimental.pallas import tpu as pltpu
from jax.experimental.pallas import tpu_sc as plsc

VOCAB = 100000
SEQ = 2048
D = 1024
B = 4
NC = 2
NS = 16
NW = NC * NS
POS_PER_W = SEQ // NW
C = 16
PB = POS_PER_W // C
NCHUNK = PB * B
NB_R = 3
NB_O = 3
SCALE = math.sqrt(D)
PE_INV = 1.0 / 127.0


def _pe_table() -> np.ndarray:
    pos = np.arange(SEQ, dtype=np.float32)[:, None]
    div = np.exp(np.arange(0, D, 2, dtype=np.float32) * (-math.log(10000.0) / D))
    pe = np.zeros((SEQ, D), np.float32)
    pe[:, 0::2] = np.sin(pos * div)
    pe[:, 1::2] = np.cos(pos * div)
    q = np.round(pe.reshape(-1) * 127.0).astype(np.int32)
    q = (q & 0xFF).reshape(-1, 4, 16).astype(np.uint32)
    words = q[:, 0, :] | (q[:, 1, :] << 8) | (q[:, 2, :] << 16) | (q[:, 3, :] << 24)
    return words.reshape(SEQ * D // 4).view(np.int32)


_PE = _pe_table()


def _embed_kernel(tok_hbm, w_hbm, pe_hbm, out_hbm, idx_v, peb, rows, outb, psem, isem, gsems, osems):
    wid = lax.axis_index("s") * NC + lax.axis_index("c")
    pos_base = wid * POS_PER_W

    pe_desc = pltpu.async_copy(
        pe_hbm.at[pl.ds(pos_base * (D // 4), POS_PER_W * D // 4)], peb, psem)

    idx_descs = [
        pltpu.async_copy(tok_hbm.at[s, pl.ds(pos_base, POS_PER_W)],
                         idx_v.at[s], isem)
        for s in range(B)
    ]
    for d in idx_descs:
        d.wait()

    gh, oh = {}, {}

    def sp(c):
        return c % B, c // B

    def start_gather(c):
        s, pb = sp(c)
        idx = idx_v.at[s, pl.ds(pb * C, C)]
        gh[c] = pltpu.async_copy(w_hbm.at[idx], rows[c % NB_R], gsems[c % NB_R])

    def start_out(c):
        s, pb = sp(c)
        dst = out_hbm.at[s, pl.ds(pos_base + pb * C, C)]
        oh[c] = pltpu.async_copy(outb[c % NB_O], dst, osems[c % NB_O])

    def compute(c):
        rv, ov = rows[c % NB_R], outb[c % NB_O]
        _, pb = sp(c)
        pe_row0 = pb * C

        def rbody(r, carry):
            base = (pe_row0 + r) * (D // 4)

            @plsc.parallel_loop(0, D // 64, unroll=1)
            def jbody(m):
                off = m * 64
                w = peb[pl.ds(base + m * 16, 16)]
                for k in range(4):
                    x = lax.shift_right_arithmetic(
                        lax.shift_left(w, 24 - 8 * k), 24)
                    a = x.astype(jnp.float32) * PE_INV
                    v = rv[r, pl.ds(off + k * 16, 16)]
                    ov[r, pl.ds(off + k * 16, 16)] = a + v * SCALE
            return carry
        lax.fori_loop(0, C, rbody, None)

    for c in range(NB_R):
        start_gather(c)
    pe_desc.wait()

    for c in range(NCHUNK):
        if c >= NB_O:
            oh[c - NB_O].wait()
        gh[c].wait()
        compute(c)
        start_out(c)
        if c + NB_R < NCHUNK:
            start_gather(c + NB_R)
    for c in range(NCHUNK - NB_O, NCHUNK):
        oh[c].wait()


def kernel(tokens, W):
    mesh = plsc.VectorSubcoreMesh(
        core_axis_name="c", subcore_axis_name="s", num_cores=NC, num_subcores=NS
    )
    run = pl.kernel(
        _embed_kernel,
        out_type=jax.ShapeDtypeStruct((B, SEQ, D), jnp.float32),
        mesh=mesh,
        scratch_types=[
            pltpu.VMEM((B, POS_PER_W), jnp.int32),
            pltpu.VMEM((POS_PER_W * D // 4,), jnp.int32),
            [pltpu.VMEM((C, D), jnp.float32) for _ in range(NB_R)],
            [pltpu.VMEM((C, D), jnp.float32) for _ in range(NB_O)],
            pltpu.SemaphoreType.DMA,
            pltpu.SemaphoreType.DMA,
            [pltpu.SemaphoreType.DMA for _ in range(NB_R)],
            [pltpu.SemaphoreType.DMA for _ in range(NB_O)],
        ],
    )
    return run(tokens.astype(jnp.int32), W, jnp.asarray(_PE))


# speedup vs baseline: 1.0028x; 1.0028x over previous
import math

import numpy as np
import jax
import jax.numpy as jnp
from jax import lax
from jax.experimental import pallas as pl
from jax.experimental.pallas import tpu as pltpu
from jax.experimental.pallas import tpu_sc as plsc

VOCAB = 100000
SEQ = 2048
D = 1024
B = 4
NC = 2
NS = 16
NW = NC * NS
POS_PER_W = SEQ // NW
C = 16
PB = POS_PER_W // C
NCHUNK = PB * B
NB_R = 3
NB_O = 2
SCALE = math.sqrt(D)
PE_INV = 1.0 / 127.0


def _pe_table() -> np.ndarray:
    pos = np.arange(SEQ, dtype=np.float32)[:, None]
    div = np.exp(np.arange(0, D, 2, dtype=np.float32) * (-math.log(10000.0) / D))
    pe = np.zeros((SEQ, D), np.float32)
    pe[:, 0::2] = np.sin(pos * div)
    pe[:, 1::2] = np.cos(pos * div)
    q = np.round(pe.reshape(-1) * 127.0).astype(np.int32)
    q = (q & 0xFF).reshape(-1, 4, 16).astype(np.uint32)
    words = q[:, 0, :] | (q[:, 1, :] << 8) | (q[:, 2, :] << 16) | (q[:, 3, :] << 24)
    return words.reshape(SEQ * D // 4).view(np.int32)


_PE = _pe_table()


def _embed_kernel(tok_hbm, w_hbm, pe_hbm, out_hbm, idx_v, peb, rows, outb, psem, isem, gsems, osems):
    wid = lax.axis_index("s") * NC + lax.axis_index("c")
    pos_base = wid * POS_PER_W

    pe_desc = pltpu.async_copy(
        pe_hbm.at[pl.ds(pos_base * (D // 4), POS_PER_W * D // 4)], peb, psem)

    idx_descs = [
        pltpu.async_copy(tok_hbm.at[s, pl.ds(pos_base, POS_PER_W)],
                         idx_v.at[s], isem)
        for s in range(B)
    ]
    for d in idx_descs:
        d.wait()

    gh, oh = {}, {}

    def sp(c):
        return c % B, c // B

    def start_gather(c):
        s, pb = sp(c)
        idx = idx_v.at[s, pl.ds(pb * C, C)]
        gh[c] = pltpu.async_copy(w_hbm.at[idx], rows[c % NB_R], gsems[c % NB_R])

    def start_out(c):
        s, pb = sp(c)
        dst = out_hbm.at[s, pl.ds(pos_base + pb * C, C)]
        oh[c] = pltpu.async_copy(outb[c % NB_O], dst, osems[c % NB_O])

    def compute(c):
        rv, ov = rows[c % NB_R], outb[c % NB_O]
        _, pb = sp(c)
        pe_row0 = pb * C

        def rbody(r, carry):
            base = (pe_row0 + r) * (D // 4)

            @plsc.parallel_loop(0, D // 64, unroll=1)
            def jbody(m):
                off = m * 64
                w = peb[pl.ds(base + m * 16, 16)]
                for k in range(4):
                    x = lax.shift_right_arithmetic(
                        lax.shift_left(w, 24 - 8 * k), 24)
                    a = x.astype(jnp.float32) * PE_INV
                    v = rv[r, pl.ds(off + k * 16, 16)]
                    ov[r, pl.ds(off + k * 16, 16)] = a + v * SCALE
            return carry
        lax.fori_loop(0, C, rbody, None)

    for c in range(NB_R):
        start_gather(c)
    pe_desc.wait()

    for c in range(NCHUNK):
        if c >= NB_O:
            oh[c - NB_O].wait()
        gh[c].wait()
        compute(c)
        start_out(c)
        if c + NB_R < NCHUNK:
            start_gather(c + NB_R)
    for c in range(NCHUNK - NB_O, NCHUNK):
        oh[c].wait()


def kernel(tokens, W):
    mesh = plsc.VectorSubcoreMesh(
        core_axis_name="c", subcore_axis_name="s", num_cores=NC, num_subcores=NS
    )
    run = pl.kernel(
        _embed_kernel,
        out_type=jax.ShapeDtypeStruct((B, SEQ, D), jnp.float32),
        mesh=mesh,
        scratch_types=[
            pltpu.VMEM((B, POS_PER_W), jnp.int32),
            pltpu.VMEM((POS_PER_W * D // 4,), jnp.int32),
            [pltpu.VMEM((C, D), jnp.float32) for _ in range(NB_R)],
            [pltpu.VMEM((C, D), jnp.float32) for _ in range(NB_O)],
            pltpu.SemaphoreType.DMA,
            pltpu.SemaphoreType.DMA,
            [pltpu.SemaphoreType.DMA for _ in range(NB_R)],
            [pltpu.SemaphoreType.DMA for _ in range(NB_O)],
        ],
    )
    return run(tokens.astype(jnp.int32), W, jnp.asarray(_PE))
